# baseline (speedup 1.0000x reference)
import functools

import jax
import jax.numpy as jnp
from jax import lax
from jax.experimental import pallas as pl
from jax.experimental.pallas import tpu as pltpu
from jax.experimental.pallas import tpu_sc as plsc

N = 10000
E = 320000
D_IN = 128
P = 32
K2 = 9

_info = plsc.get_sparse_core_info()
NC = _info.num_cores
NS = _info.num_subcores
LN = _info.num_lanes
NW = NC * NS
EW = E // NW
CH = 80
NCH = -(-EW // CH)
EP = NW * NCH * CH
NP = 10240
RPT = NP // NS


def _widx():
    return lax.axis_index("c") * NS + lax.axis_index("s")


def _segsum(table, src3, dst3, d):
    mesh = plsc.VectorSubcoreMesh(core_axis_name="c", subcore_axis_name="s")

    @functools.partial(
        pl.kernel,
        mesh=mesh,
        compiler_params=pltpu.CompilerParams(use_tc_tiling_on_sc=False),
        out_type=jax.ShapeDtypeStruct((NC, NP, d), jnp.float32),
        scratch_types=[
            pltpu.VMEM((NCH + 3, CH), jnp.int32),
            pltpu.VMEM((NCH, CH), jnp.int32),
            pltpu.VMEM((4, CH, d), jnp.float32),
            pltpu.VMEM((RPT, d), jnp.float32),
            pltpu.VMEM_SHARED((NP, d), jnp.float32),
            [pltpu.SemaphoreType.DMA] * 4,
        ],
    )
    def k(table_h, src_h, dst_h, out_h, sidx, didx, rows, zbuf, acc, sems):
        cid = lax.axis_index("c")
        sid = lax.axis_index("s")
        wid = _widx()
        pltpu.sync_copy(src_h.at[wid], sidx.at[pl.ds(0, NCH)])
        pltpu.sync_copy(dst_h.at[wid], didx)

        z16 = jnp.zeros((LN,), jnp.float32)
        zi16 = jnp.zeros((LN,), jnp.int32)
        for r in range(NCH, NCH + 3):
            for g in range(CH // LN):
                sidx[r, pl.ds(g * LN, LN)] = zi16

        def zero_row(i, carry):
            for h in range(-(-d // LN)):
                st = min(h * LN, d - LN)
                zbuf[i, pl.ds(st, LN)] = z16
            return carry

        lax.fori_loop(0, RPT, zero_row, 0)
        pltpu.sync_copy(zbuf, acc.at[pl.ds(sid * RPT, RPT)])
        plsc.subcore_barrier()

        def gath(j, u):
            pltpu.async_copy(table_h.at[sidx.at[j]], rows.at[u], sems[u])

        def gwait(j, u):
            pltpu.make_async_copy(table_h.at[sidx.at[j]], rows.at[u],
                                  sems[u]).wait()

        gath(0, 0)

        def pair(jp, carry):
            j0 = 2 * jp
            j1 = j0 + 1
            gwait(j0, 0)
            gath(j1, 1)
            pltpu.sync_copy(rows.at[0], acc.at[didx.at[j0]], add=True)
            gwait(j1, 1)
            gath(j0 + 2, 0)
            pltpu.sync_copy(rows.at[1], acc.at[didx.at[j1]], add=True)
            return carry

        lax.fori_loop(0, (NCH - 1) // 2, pair, 0)
        gwait(NCH - 1, 0)
        pltpu.sync_copy(rows.at[0], acc.at[didx.at[NCH - 1]], add=True)
        plsc.subcore_barrier()
        pltpu.sync_copy(acc.at[pl.ds(sid * RPT, RPT)], zbuf)
        pltpu.sync_copy(zbuf, out_h.at[cid, pl.ds(sid * RPT, RPT)])

    return k(table, src3, dst3)


def _spline_agg(hw, src3, dst3, f03, f13):
    mesh = plsc.VectorSubcoreMesh(core_axis_name="c", subcore_axis_name="s")

    @functools.partial(
        pl.kernel,
        mesh=mesh,
        compiler_params=pltpu.CompilerParams(use_tc_tiling_on_sc=False),
        out_type=jax.ShapeDtypeStruct((NC, NP, P), jnp.float32),
        scratch_types=[
            pltpu.VMEM((NCH + 3, CH), jnp.int32),
            pltpu.VMEM((NCH, CH), jnp.int32),
            pltpu.VMEM((NCH + 3, CH), jnp.float32),
            pltpu.VMEM((NCH + 3, CH), jnp.float32),
            pltpu.VMEM((2, CH), jnp.int32),
            pltpu.VMEM((2, 4, CH), jnp.float32),
            pltpu.VMEM((2, CH, 4 * P), jnp.float32),
            pltpu.VMEM((CH, P), jnp.float32),
            pltpu.VMEM((RPT, P), jnp.float32),
            pltpu.VMEM_SHARED((NP, P), jnp.float32),
            [pltpu.SemaphoreType.DMA] * 4,
        ],
    )
    def k(hw_h, src_h, dst_h, f0_h, f1_h, out_h,
          sidx, didx, fa, fb, gidx, wbuf, rbuf, msg, zbuf, acc, sems):
        cid = lax.axis_index("c")
        sid = lax.axis_index("s")
        wid = _widx()
        pltpu.sync_copy(src_h.at[wid], sidx.at[pl.ds(0, NCH)])
        pltpu.sync_copy(dst_h.at[wid], didx)
        pltpu.sync_copy(f0_h.at[wid], fa.at[pl.ds(0, NCH)])
        pltpu.sync_copy(f1_h.at[wid], fb.at[pl.ds(0, NCH)])

        z16 = jnp.zeros((LN,), jnp.float32)
        zi16 = jnp.zeros((LN,), jnp.int32)
        for r in range(NCH, NCH + 3):
            for g in range(CH // LN):
                sidx[r, pl.ds(g * LN, LN)] = zi16
                fa[r, pl.ds(g * LN, LN)] = z16
                fb[r, pl.ds(g * LN, LN)] = z16

        def zero_row(i, carry):
            for h in range(P // LN):
                zbuf[i, pl.ds(h * LN, LN)] = z16
            return carry

        lax.fori_loop(0, RPT, zero_row, 0)
        pltpu.sync_copy(zbuf, acc.at[pl.ds(sid * RPT, RPT)])
        plsc.subcore_barrier()

        def weights(j, b):
            for g in range(CH // LN):
                sl = pl.ds(g * LN, LN)
                s = sidx[j, sl]
                va = fa[j, sl] * 2.0
                ia = va.astype(jnp.int32)
                fra = va - ia.astype(jnp.float32)
                vb = fb[j, sl] * 2.0
                ib = vb.astype(jnp.int32)
                frb = vb - ib.astype(jnp.float32)
                gidx[b, sl] = s * 4 + ia + 2 * ib
                for b1 in (0, 1):
                    wb1 = frb if b1 else 1.0 - frb
                    for b0 in (0, 1):
                        jj = b0 + 2 * b1
                        wa = fra if b0 else 1.0 - fra
                        wbuf[b, jj, sl] = wa * wb1

        def fire(b):
            pltpu.async_copy(hw_h.at[gidx.at[b]], rbuf.at[b], sems[b])

        def drain(b):
            pltpu.make_async_copy(hw_h.at[gidx.at[b]], rbuf.at[b],
                                  sems[b]).wait()

        def combine_scatter(j, b):
            def comb(g, c2):
                gsl = pl.ds(g * LN, LN)
                wrows = [wbuf[b, jj, gsl] for jj in range(4)]
                for li in range(LN):
                    i = g * LN + li
                    lidx = jnp.full((LN,), li, jnp.int32)
                    ws = [jnp.take_along_axis(wrows[jj], lidx, axis=0)
                          for jj in range(4)]
                    for h in range(P // LN):
                        sl = pl.ds(h * LN, LN)
                        v = ws[0] * rbuf[b, i, pl.ds(h * LN, LN)]
                        for jj in range(1, 4):
                            v = v + ws[jj] * rbuf[b, i,
                                                  pl.ds(jj * P + h * LN, LN)]
                        msg[i, sl] = v
                return c2

            lax.fori_loop(0, CH // LN, comb, 0)
            pltpu.sync_copy(msg, acc.at[didx.at[j]], add=True)

        weights(0, 0)
        fire(0)

        def pair(jp, carry):
            j0 = 2 * jp
            j1 = j0 + 1
            weights(j1, 1)
            drain(0)
            fire(1)
            combine_scatter(j0, 0)
            weights(j0 + 2, 0)
            drain(1)
            fire(0)
            combine_scatter(j1, 1)
            return carry

        lax.fori_loop(0, (NCH - 1) // 2, pair, 0)
        drain(0)
        combine_scatter(NCH - 1, 0)
        plsc.subcore_barrier()
        pltpu.sync_copy(acc.at[pl.ds(sid * RPT, RPT)], zbuf)
        pltpu.sync_copy(zbuf, out_h.at[cid, pl.ds(sid * RPT, RPT)])

    return k(hw, src3, dst3, f03, f13)


def _bn(z, g, b):
    mean = jnp.mean(z, axis=0, keepdims=True)
    var = jnp.mean((z - mean) * (z - mean), axis=0, keepdims=True)
    return (z - mean) / jnp.sqrt(var + 1e-5) * g + b


def _tc_stage1(x, w1, r1w):
    def body(x_ref, w1_ref, r1w_ref, t1_ref, r1_ref):
        xv = x_ref[...]
        t1_ref[:, :P] = jnp.dot(xv, w1_ref[...],
                                preferred_element_type=jnp.float32)
        t1_ref[:, P:] = jnp.ones((N, 8), jnp.float32)
        r1_ref[...] = jnp.dot(xv, r1w_ref[...],
                              preferred_element_type=jnp.float32)

    return pl.pallas_call(
        body,
        out_shape=[jax.ShapeDtypeStruct((N, P + 8), jnp.float32),
                   jax.ShapeDtypeStruct((N, P), jnp.float32)],
    )(x, w1, r1w)


def _tc_stage2(psum1, r1, b1, g1, be1, w2m, r2w):
    def body(p_ref, r1_ref, b1_ref, g1_ref, be1_ref, w2m_ref, r2w_ref,
             hw_ref, r2_ref, crec_ref):
        p = p_ref[0] + p_ref[1]
        c = jnp.maximum(p[:, P:P + 1], 1.0)
        z = p[:, :P] / c + r1_ref[...] + b1_ref[...]
        h1 = jax.nn.relu(_bn(z, g1_ref[...], be1_ref[...]))
        hw_ref[...] = jnp.dot(h1, w2m_ref[...],
                              preferred_element_type=jnp.float32)
        r2_ref[...] = jnp.dot(h1, r2w_ref[...],
                              preferred_element_type=jnp.float32)
        crec_ref[...] = jnp.broadcast_to(1.0 / c, (N, 8))

    return pl.pallas_call(
        body,
        out_shape=[jax.ShapeDtypeStruct((N, 16 * P), jnp.float32),
                   jax.ShapeDtypeStruct((N, P), jnp.float32),
                   jax.ShapeDtypeStruct((N, 8), jnp.float32)],
    )(psum1, r1, b1, g1, be1, w2m, r2w)


def _tc_stage3(psum2, r2, crec, b2, g2, be2):
    def body(p_ref, r2_ref, crec_ref, b2_ref, g2_ref, be2_ref, h2_ref):
        s = p_ref[0] + p_ref[1]
        z = s * crec_ref[:, :1] + r2_ref[...] + b2_ref[...]
        h2_ref[...] = jax.nn.relu(_bn(z, g2_ref[...], be2_ref[...]))

    return pl.pallas_call(
        body,
        out_shape=jax.ShapeDtypeStruct((N, P), jnp.float32),
    )(psum2, r2, crec, b2, g2, be2)


def _tc_stage4(psum3, crec, h2, x, w3, r3w, b3, g3, be3):
    def body(p_ref, crec_ref, h2_ref, x_ref, w3_ref, r3w_ref,
             b3_ref, g3_ref, be3_ref, out_ref):
        agg = (p_ref[0] + p_ref[1]) * crec_ref[:, :1]
        z = (jnp.dot(agg, w3_ref[...], preferred_element_type=jnp.float32)
             + jnp.dot(h2_ref[...], r3w_ref[...],
                       preferred_element_type=jnp.float32)
             + b3_ref[...])
        out_ref[...] = jax.nn.relu(_bn(z, g3_ref[...], be3_ref[...])
                                   + x_ref[...])

    return pl.pallas_call(
        body,
        out_shape=jax.ShapeDtypeStruct((N, D_IN), jnp.float32),
    )(psum3, crec, h2, x, w3, r3w, b3, g3, be3)


@jax.jit
def kernel(x, edge_index, edge_attr, W1, R1, b1, g1, be1,
           W2, R2, b2, g2, be2, W3, R3, b3, g3, be3):
    pad = EP - E
    src3 = jnp.pad(edge_index[0], (0, pad)).reshape(NW, NCH, CH)
    dst3 = jnp.pad(edge_index[1], (0, pad),
                   constant_values=N).reshape(NW, NCH, CH)
    f03 = jnp.pad(edge_attr[:, 0], (0, pad)).reshape(NW, NCH, CH)
    f13 = jnp.pad(edge_attr[:, 1], (0, pad)).reshape(NW, NCH, CH)
    w2m = jnp.concatenate(
        [W2[(i0 + b0) + 3 * (i1 + b1)]
         for i1 in (0, 1) for i0 in (0, 1)
         for b1 in (0, 1) for b0 in (0, 1)], axis=1)

    b1r = b1.reshape(1, P)
    g1r = g1.reshape(1, P)
    be1r = be1.reshape(1, P)
    b2r = b2.reshape(1, P)
    g2r = g2.reshape(1, P)
    be2r = be2.reshape(1, P)
    b3r = b3.reshape(1, D_IN)
    g3r = g3.reshape(1, D_IN)
    be3r = be3.reshape(1, D_IN)

    t1, r1 = _tc_stage1(x, W1[0], R1)
    psum1 = _segsum(t1, src3, dst3, P + 8)[:, :N]
    hw, r2, crec = _tc_stage2(psum1, r1, b1r, g1r, be1r, w2m, R2)

    hw_flat = hw.reshape(N * 4, 4 * P)
    psum2 = _spline_agg(hw_flat, src3, dst3, f03, f13)[:, :N]
    h2 = _tc_stage3(psum2, r2, crec, b2r, g2r, be2r)

    psum3 = _segsum(h2, src3, dst3, P)[:, :N]
    return _tc_stage4(psum3, crec, h2, x, W3[0], R3, b3r, g3r, be3r)


# speedup vs baseline: 1.1344x; 1.1344x over previous
import functools

import jax
import jax.numpy as jnp
from jax import lax
from jax.experimental import pallas as pl
from jax.experimental.pallas import tpu as pltpu
from jax.experimental.pallas import tpu_sc as plsc

N = 10000
E = 320000
D_IN = 128
P = 32
K2 = 9

_info = plsc.get_sparse_core_info()
NC = _info.num_cores
NS = _info.num_subcores
LN = _info.num_lanes
NW = NC * NS
EW = E // NW
CH = 80
NCH = -(-EW // CH)
EP = NW * NCH * CH
NP = 10240
RPT = NP // NS


def _widx():
    return lax.axis_index("c") * NS + lax.axis_index("s")


def _segsum(table, src3, dst3, d):
    mesh = plsc.VectorSubcoreMesh(core_axis_name="c", subcore_axis_name="s")

    @functools.partial(
        pl.kernel,
        mesh=mesh,
        compiler_params=pltpu.CompilerParams(use_tc_tiling_on_sc=False),
        out_type=jax.ShapeDtypeStruct((NC, NP, d), jnp.float32),
        scratch_types=[
            pltpu.VMEM((NCH + 3, CH), jnp.int32),
            pltpu.VMEM((NCH, CH), jnp.int32),
            pltpu.VMEM((4, CH, d), jnp.float32),
            pltpu.VMEM((RPT, d), jnp.float32),
            pltpu.VMEM_SHARED((NP, d), jnp.float32),
            [pltpu.SemaphoreType.DMA] * 4,
        ],
    )
    def k(table_h, src_h, dst_h, out_h, sidx, didx, rows, zbuf, acc, sems):
        cid = lax.axis_index("c")
        sid = lax.axis_index("s")
        wid = _widx()
        pltpu.sync_copy(src_h.at[wid], sidx.at[pl.ds(0, NCH)])
        pltpu.sync_copy(dst_h.at[wid], didx)

        z16 = jnp.zeros((LN,), jnp.float32)
        zi16 = jnp.zeros((LN,), jnp.int32)
        for r in range(NCH, NCH + 3):
            for g in range(CH // LN):
                sidx[r, pl.ds(g * LN, LN)] = zi16

        def zero_row(i, carry):
            for h in range(-(-d // LN)):
                st = min(h * LN, d - LN)
                zbuf[i, pl.ds(st, LN)] = z16
            return carry

        lax.fori_loop(0, RPT, zero_row, 0)
        pltpu.sync_copy(zbuf, acc.at[pl.ds(sid * RPT, RPT)])
        plsc.subcore_barrier()

        def gath(j, u):
            pltpu.async_copy(table_h.at[sidx.at[j]], rows.at[u], sems[u])

        def gwait(j, u):
            pltpu.make_async_copy(table_h.at[sidx.at[j]], rows.at[u],
                                  sems[u]).wait()

        gath(0, 0)

        def pair(jp, carry):
            j0 = 2 * jp
            j1 = j0 + 1
            gwait(j0, 0)
            gath(j1, 1)
            pltpu.sync_copy(rows.at[0], acc.at[didx.at[j0]], add=True)
            gwait(j1, 1)
            gath(j0 + 2, 0)
            pltpu.sync_copy(rows.at[1], acc.at[didx.at[j1]], add=True)
            return carry

        lax.fori_loop(0, (NCH - 1) // 2, pair, 0)
        gwait(NCH - 1, 0)
        pltpu.sync_copy(rows.at[0], acc.at[didx.at[NCH - 1]], add=True)
        plsc.subcore_barrier()
        pltpu.sync_copy(acc.at[pl.ds(sid * RPT, RPT)], zbuf)
        pltpu.sync_copy(zbuf, out_h.at[cid, pl.ds(sid * RPT, RPT)])

    return k(table, src3, dst3)


def _spline_agg(hw, src3, dst3, f03, f13):
    mesh = plsc.VectorSubcoreMesh(core_axis_name="c", subcore_axis_name="s")

    @functools.partial(
        pl.kernel,
        mesh=mesh,
        compiler_params=pltpu.CompilerParams(use_tc_tiling_on_sc=False),
        out_type=jax.ShapeDtypeStruct((NC, NP, P), jnp.float32),
        scratch_types=[
            pltpu.VMEM((NCH + 3, CH), jnp.int32),
            pltpu.VMEM((NCH, CH), jnp.int32),
            pltpu.VMEM((NCH + 3, CH), jnp.float32),
            pltpu.VMEM((NCH + 3, CH), jnp.float32),
            pltpu.VMEM((2, 4, CH), jnp.int32),
            pltpu.VMEM((2, 4, CH), jnp.float32),
            pltpu.VMEM((2, 4, CH, P), jnp.float32),
            pltpu.VMEM((CH, P), jnp.float32),
            pltpu.VMEM((RPT, P), jnp.float32),
            pltpu.VMEM_SHARED((NP, P), jnp.float32),
            [pltpu.SemaphoreType.DMA] * 4,
        ],
    )
    def k(hw_h, src_h, dst_h, f0_h, f1_h, out_h,
          sidx, didx, fa, fb, gidx, wbuf, rbuf, msg, zbuf, acc, sems):
        cid = lax.axis_index("c")
        sid = lax.axis_index("s")
        wid = _widx()
        pltpu.sync_copy(src_h.at[wid], sidx.at[pl.ds(0, NCH)])
        pltpu.sync_copy(dst_h.at[wid], didx)
        pltpu.sync_copy(f0_h.at[wid], fa.at[pl.ds(0, NCH)])
        pltpu.sync_copy(f1_h.at[wid], fb.at[pl.ds(0, NCH)])

        z16 = jnp.zeros((LN,), jnp.float32)
        zi16 = jnp.zeros((LN,), jnp.int32)
        for r in range(NCH, NCH + 3):
            for g in range(CH // LN):
                sidx[r, pl.ds(g * LN, LN)] = zi16
                fa[r, pl.ds(g * LN, LN)] = z16
                fb[r, pl.ds(g * LN, LN)] = z16

        def zero_row(i, carry):
            for h in range(P // LN):
                zbuf[i, pl.ds(h * LN, LN)] = z16
            return carry

        lax.fori_loop(0, RPT, zero_row, 0)
        pltpu.sync_copy(zbuf, acc.at[pl.ds(sid * RPT, RPT)])
        plsc.subcore_barrier()

        def weights(j, b):
            for g in range(CH // LN):
                sl = pl.ds(g * LN, LN)
                s = sidx[j, sl]
                va = fa[j, sl] * 2.0
                ia = va.astype(jnp.int32)
                fra = va - ia.astype(jnp.float32)
                vb = fb[j, sl] * 2.0
                ib = vb.astype(jnp.int32)
                frb = vb - ib.astype(jnp.float32)
                base = s * 9 + ia + ib * 3
                for b1 in (0, 1):
                    wb1 = frb if b1 else 1.0 - frb
                    for b0 in (0, 1):
                        jj = b0 + 2 * b1
                        gidx[b, jj, sl] = base + (b0 + 3 * b1)
                        wa = fra if b0 else 1.0 - fra
                        wbuf[b, jj, sl] = wa * wb1

        def fire(b):
            for jj in range(4):
                pltpu.async_copy(hw_h.at[gidx.at[b, jj]], rbuf.at[b, jj],
                                 sems[b])

        def drain(b):
            for jj in range(4):
                pltpu.make_async_copy(hw_h.at[gidx.at[b, jj]],
                                      rbuf.at[b, jj], sems[b]).wait()

        def combine_scatter(j, b):
            def comb(g, c2):
                gsl = pl.ds(g * LN, LN)
                wrows = [wbuf[b, jj, gsl] for jj in range(4)]
                for li in range(LN):
                    i = g * LN + li
                    lidx = jnp.full((LN,), li, jnp.int32)
                    ws = [jnp.take_along_axis(wrows[jj], lidx, axis=0)
                          for jj in range(4)]
                    for h in range(P // LN):
                        sl = pl.ds(h * LN, LN)
                        v = ws[0] * rbuf[b, 0, i, sl]
                        for jj in range(1, 4):
                            v = v + ws[jj] * rbuf[b, jj, i, sl]
                        msg[i, sl] = v
                return c2

            lax.fori_loop(0, CH // LN, comb, 0)
            pltpu.sync_copy(msg, acc.at[didx.at[j]], add=True)

        weights(0, 0)
        fire(0)

        def pair(jp, carry):
            j0 = 2 * jp
            j1 = j0 + 1
            weights(j1, 1)
            drain(0)
            fire(1)
            combine_scatter(j0, 0)
            weights(j0 + 2, 0)
            drain(1)
            fire(0)
            combine_scatter(j1, 1)
            return carry

        lax.fori_loop(0, (NCH - 1) // 2, pair, 0)
        drain(0)
        combine_scatter(NCH - 1, 0)
        plsc.subcore_barrier()
        pltpu.sync_copy(acc.at[pl.ds(sid * RPT, RPT)], zbuf)
        pltpu.sync_copy(zbuf, out_h.at[cid, pl.ds(sid * RPT, RPT)])

    return k(hw, src3, dst3, f03, f13)


def _bn(z, g, b):
    mean = jnp.mean(z, axis=0, keepdims=True)
    var = jnp.mean((z - mean) * (z - mean), axis=0, keepdims=True)
    return (z - mean) / jnp.sqrt(var + 1e-5) * g + b


def _tc_stage1(x, w1, r1w):
    def body(x_ref, w1_ref, r1w_ref, t1_ref, r1_ref):
        xv = x_ref[...]
        t1_ref[:, :P] = jnp.dot(xv, w1_ref[...],
                                preferred_element_type=jnp.float32)
        t1_ref[:, P:] = jnp.ones((N, 8), jnp.float32)
        r1_ref[...] = jnp.dot(xv, r1w_ref[...],
                              preferred_element_type=jnp.float32)

    return pl.pallas_call(
        body,
        out_shape=[jax.ShapeDtypeStruct((N, P + 8), jnp.float32),
                   jax.ShapeDtypeStruct((N, P), jnp.float32)],
    )(x, w1, r1w)


def _tc_stage2(psum1, r1, b1, g1, be1, w2m, r2w):
    def body(p_ref, r1_ref, b1_ref, g1_ref, be1_ref, w2m_ref, r2w_ref,
             hw_ref, r2_ref, crec_ref):
        p = p_ref[0] + p_ref[1]
        c = jnp.maximum(p[:, P:P + 1], 1.0)
        z = p[:, :P] / c + r1_ref[...] + b1_ref[...]
        h1 = jax.nn.relu(_bn(z, g1_ref[...], be1_ref[...]))
        hw_ref[...] = jnp.dot(h1, w2m_ref[...],
                              preferred_element_type=jnp.float32)
        r2_ref[...] = jnp.dot(h1, r2w_ref[...],
                              preferred_element_type=jnp.float32)
        crec_ref[...] = jnp.broadcast_to(1.0 / c, (N, 8))

    return pl.pallas_call(
        body,
        out_shape=[jax.ShapeDtypeStruct((N, K2 * P), jnp.float32),
                   jax.ShapeDtypeStruct((N, P), jnp.float32),
                   jax.ShapeDtypeStruct((N, 8), jnp.float32)],
    )(psum1, r1, b1, g1, be1, w2m, r2w)


def _tc_stage3(psum2, r2, crec, b2, g2, be2):
    def body(p_ref, r2_ref, crec_ref, b2_ref, g2_ref, be2_ref, h2_ref):
        s = p_ref[0] + p_ref[1]
        z = s * crec_ref[:, :1] + r2_ref[...] + b2_ref[...]
        h2_ref[...] = jax.nn.relu(_bn(z, g2_ref[...], be2_ref[...]))

    return pl.pallas_call(
        body,
        out_shape=jax.ShapeDtypeStruct((N, P), jnp.float32),
    )(psum2, r2, crec, b2, g2, be2)


def _tc_stage4(psum3, crec, h2, x, w3, r3w, b3, g3, be3):
    def body(p_ref, crec_ref, h2_ref, x_ref, w3_ref, r3w_ref,
             b3_ref, g3_ref, be3_ref, out_ref):
        agg = (p_ref[0] + p_ref[1]) * crec_ref[:, :1]
        z = (jnp.dot(agg, w3_ref[...], preferred_element_type=jnp.float32)
             + jnp.dot(h2_ref[...], r3w_ref[...],
                       preferred_element_type=jnp.float32)
             + b3_ref[...])
        out_ref[...] = jax.nn.relu(_bn(z, g3_ref[...], be3_ref[...])
                                   + x_ref[...])

    return pl.pallas_call(
        body,
        out_shape=jax.ShapeDtypeStruct((N, D_IN), jnp.float32),
    )(psum3, crec, h2, x, w3, r3w, b3, g3, be3)


@jax.jit
def kernel(x, edge_index, edge_attr, W1, R1, b1, g1, be1,
           W2, R2, b2, g2, be2, W3, R3, b3, g3, be3):
    pad = EP - E
    src3 = jnp.pad(edge_index[0], (0, pad)).reshape(NW, NCH, CH)
    dst3 = jnp.pad(edge_index[1], (0, pad),
                   constant_values=N).reshape(NW, NCH, CH)
    f03 = jnp.pad(edge_attr[:, 0], (0, pad)).reshape(NW, NCH, CH)
    f13 = jnp.pad(edge_attr[:, 1], (0, pad)).reshape(NW, NCH, CH)
    w2m = jnp.transpose(W2, (1, 0, 2)).reshape(P, K2 * P)

    b1r = b1.reshape(1, P)
    g1r = g1.reshape(1, P)
    be1r = be1.reshape(1, P)
    b2r = b2.reshape(1, P)
    g2r = g2.reshape(1, P)
    be2r = be2.reshape(1, P)
    b3r = b3.reshape(1, D_IN)
    g3r = g3.reshape(1, D_IN)
    be3r = be3.reshape(1, D_IN)

    t1, r1 = _tc_stage1(x, W1[0], R1)
    psum1 = _segsum(t1, src3, dst3, P + 8)[:, :N]
    hw, r2, crec = _tc_stage2(psum1, r1, b1r, g1r, be1r, w2m, R2)

    hw_flat = hw.reshape(N * K2, P)
    psum2 = _spline_agg(hw_flat, src3, dst3, f03, f13)[:, :N]
    h2 = _tc_stage3(psum2, r2, crec, b2r, g2r, be2r)

    psum3 = _segsum(h2, src3, dst3, P)[:, :N]
    return _tc_stage4(psum3, crec, h2, x, W3[0], R3, b3r, g3r, be3r)


# speedup vs baseline: 1.1837x; 1.0435x over previous
import functools

import jax
import jax.numpy as jnp
from jax import lax
from jax.experimental import pallas as pl
from jax.experimental.pallas import tpu as pltpu
from jax.experimental.pallas import tpu_sc as plsc

N = 10000
E = 320000
D_IN = 128
P = 32
K2 = 9

_info = plsc.get_sparse_core_info()
NC = _info.num_cores
NS = _info.num_subcores
LN = _info.num_lanes
NW = NC * NS
EW = E // NW
CH = 80
NCH = -(-EW // CH)
EP = NW * NCH * CH
NP = 10240
RPT = NP // NS


def _widx():
    return lax.axis_index("c") * NS + lax.axis_index("s")


def _segsum(table, src3, dst3, d):
    mesh = plsc.VectorSubcoreMesh(core_axis_name="c", subcore_axis_name="s")

    @functools.partial(
        pl.kernel,
        mesh=mesh,
        compiler_params=pltpu.CompilerParams(use_tc_tiling_on_sc=False),
        out_type=jax.ShapeDtypeStruct((NC, NP, d), jnp.float32),
        scratch_types=[
            pltpu.VMEM((NCH + 3, CH), jnp.int32),
            pltpu.VMEM((NCH, CH), jnp.int32),
            pltpu.VMEM((4, CH, d), jnp.float32),
            pltpu.VMEM((RPT, d), jnp.float32),
            pltpu.VMEM_SHARED((NP, d), jnp.float32),
            [pltpu.SemaphoreType.DMA] * 4,
        ],
    )
    def k(table_h, src_h, dst_h, out_h, sidx, didx, rows, zbuf, acc, sems):
        cid = lax.axis_index("c")
        sid = lax.axis_index("s")
        wid = _widx()
        pltpu.sync_copy(src_h.at[wid], sidx.at[pl.ds(0, NCH)])
        pltpu.sync_copy(dst_h.at[wid], didx)

        z16 = jnp.zeros((LN,), jnp.float32)
        zi16 = jnp.zeros((LN,), jnp.int32)
        for r in range(NCH, NCH + 3):
            for g in range(CH // LN):
                sidx[r, pl.ds(g * LN, LN)] = zi16

        def zero_row(i, carry):
            for h in range(-(-d // LN)):
                st = min(h * LN, d - LN)
                zbuf[i, pl.ds(st, LN)] = z16
            return carry

        lax.fori_loop(0, RPT, zero_row, 0)
        pltpu.sync_copy(zbuf, acc.at[pl.ds(sid * RPT, RPT)])
        plsc.subcore_barrier()

        def gath(j, u):
            pltpu.async_copy(table_h.at[sidx.at[j]], rows.at[u], sems[u])

        def gwait(j, u):
            pltpu.make_async_copy(table_h.at[sidx.at[j]], rows.at[u],
                                  sems[u]).wait()

        gath(0, 0)

        def pair(jp, carry):
            j0 = 2 * jp
            j1 = j0 + 1
            gwait(j0, 0)
            gath(j1, 1)
            pltpu.sync_copy(rows.at[0], acc.at[didx.at[j0]], add=True)
            gwait(j1, 1)
            gath(j0 + 2, 0)
            pltpu.sync_copy(rows.at[1], acc.at[didx.at[j1]], add=True)
            return carry

        lax.fori_loop(0, (NCH - 1) // 2, pair, 0)
        gwait(NCH - 1, 0)
        pltpu.sync_copy(rows.at[0], acc.at[didx.at[NCH - 1]], add=True)
        plsc.subcore_barrier()
        pltpu.sync_copy(acc.at[pl.ds(sid * RPT, RPT)], zbuf)
        pltpu.sync_copy(zbuf, out_h.at[cid, pl.ds(sid * RPT, RPT)])

    return k(table, src3, dst3)


def _spline_agg(hw, src3, dst3, f03, f13):
    mesh = plsc.VectorSubcoreMesh(core_axis_name="c", subcore_axis_name="s")

    @functools.partial(
        pl.kernel,
        mesh=mesh,
        compiler_params=pltpu.CompilerParams(use_tc_tiling_on_sc=False),
        out_type=jax.ShapeDtypeStruct((NC, NP, P), jnp.float32),
        scratch_types=[
            pltpu.VMEM((NCH + 3, CH), jnp.int32),
            pltpu.VMEM((NCH, CH), jnp.int32),
            pltpu.VMEM((NCH + 3, CH), jnp.float32),
            pltpu.VMEM((NCH + 3, CH), jnp.float32),
            pltpu.VMEM((2, 4, CH), jnp.int32),
            pltpu.VMEM((2, 4, CH), jnp.float32),
            pltpu.VMEM((2, 4, CH, P), jnp.float32),
            pltpu.VMEM((CH, P), jnp.float32),
            pltpu.VMEM((RPT, P), jnp.float32),
            pltpu.VMEM_SHARED((NP, P), jnp.float32),
            [pltpu.SemaphoreType.DMA] * 4,
        ],
    )
    def k(hw_h, src_h, dst_h, f0_h, f1_h, out_h,
          sidx, didx, fa, fb, gidx, wbuf, rbuf, msg, zbuf, acc, sems):
        cid = lax.axis_index("c")
        sid = lax.axis_index("s")
        wid = _widx()
        pltpu.sync_copy(src_h.at[wid], sidx.at[pl.ds(0, NCH)])
        pltpu.sync_copy(dst_h.at[wid], didx)
        pltpu.sync_copy(f0_h.at[wid], fa.at[pl.ds(0, NCH)])
        pltpu.sync_copy(f1_h.at[wid], fb.at[pl.ds(0, NCH)])

        z16 = jnp.zeros((LN,), jnp.float32)
        zi16 = jnp.zeros((LN,), jnp.int32)
        for r in range(NCH, NCH + 3):
            for g in range(CH // LN):
                sidx[r, pl.ds(g * LN, LN)] = zi16
                fa[r, pl.ds(g * LN, LN)] = z16
                fb[r, pl.ds(g * LN, LN)] = z16

        def zero_row(i, carry):
            for h in range(P // LN):
                zbuf[i, pl.ds(h * LN, LN)] = z16
            return carry

        lax.fori_loop(0, RPT, zero_row, 0)
        pltpu.sync_copy(zbuf, acc.at[pl.ds(sid * RPT, RPT)])
        plsc.subcore_barrier()

        def weights(j, b):
            for g in range(CH // LN):
                sl = pl.ds(g * LN, LN)
                s = sidx[j, sl]
                va = fa[j, sl] * 2.0
                ia = va.astype(jnp.int32)
                fra = va - ia.astype(jnp.float32)
                vb = fb[j, sl] * 2.0
                ib = vb.astype(jnp.int32)
                frb = vb - ib.astype(jnp.float32)
                base = s * 9 + ia + ib * 3
                for b1 in (0, 1):
                    wb1 = frb if b1 else 1.0 - frb
                    for b0 in (0, 1):
                        jj = b0 + 2 * b1
                        gidx[b, jj, sl] = base + (b0 + 3 * b1)
                        wa = fra if b0 else 1.0 - fra
                        wbuf[b, jj, sl] = wa * wb1

        def fire(b):
            for jj in range(4):
                pltpu.async_copy(hw_h.at[gidx.at[b, jj]], rbuf.at[b, jj],
                                 sems[b])

        def drain(b):
            for jj in range(4):
                pltpu.make_async_copy(hw_h.at[gidx.at[b, jj]],
                                      rbuf.at[b, jj], sems[b]).wait()

        def combine_scatter(j, b):
            def comb(g, c2):
                gsl = pl.ds(g * LN, LN)
                wrows = [wbuf[b, jj, gsl] for jj in range(4)]
                for li in range(LN):
                    i = g * LN + li
                    lidx = jnp.full((LN,), li, jnp.int32)
                    ws = [jnp.take_along_axis(wrows[jj], lidx, axis=0)
                          for jj in range(4)]
                    for h in range(P // LN):
                        sl = pl.ds(h * LN, LN)
                        v = ws[0] * rbuf[b, 0, i, sl]
                        for jj in range(1, 4):
                            v = v + ws[jj] * rbuf[b, jj, i, sl]
                        msg[i, sl] = v
                return c2

            lax.fori_loop(0, CH // LN, comb, 0)
            pltpu.sync_copy(msg, acc.at[didx.at[j]], add=True)

        weights(0, 0)
        fire(0)

        def pair(jp, carry):
            j0 = 2 * jp
            j1 = j0 + 1
            weights(j1, 1)
            drain(0)
            fire(1)
            combine_scatter(j0, 0)
            weights(j0 + 2, 0)
            drain(1)
            fire(0)
            combine_scatter(j1, 1)
            return carry

        lax.fori_loop(0, (NCH - 1) // 2, pair, 0)
        drain(0)
        combine_scatter(NCH - 1, 0)
        plsc.subcore_barrier()
        pltpu.sync_copy(acc.at[pl.ds(sid * RPT, RPT)], zbuf)
        pltpu.sync_copy(zbuf, out_h.at[cid, pl.ds(sid * RPT, RPT)])

    return k(hw, src3, dst3, f03, f13)


def _bn(z, g, b):
    mean = jnp.mean(z, axis=0, keepdims=True)
    var = jnp.mean((z - mean) * (z - mean), axis=0, keepdims=True)
    return (z - mean) / jnp.sqrt(var + 1e-5) * g + b


def _tc_stage1(x, w1, r1w):
    def body(x_ref, w1_ref, r1w_ref, t1_ref, r1_ref):
        xv = x_ref[...]
        t1_ref[:, :P] = jnp.dot(xv, w1_ref[...],
                                preferred_element_type=jnp.float32)
        t1_ref[:, P:] = jnp.ones((N, 8), jnp.float32)
        r1_ref[...] = jnp.dot(xv, r1w_ref[...],
                              preferred_element_type=jnp.float32)

    return pl.pallas_call(
        body,
        out_shape=[jax.ShapeDtypeStruct((N, P + 8), jnp.float32),
                   jax.ShapeDtypeStruct((N, P), jnp.float32)],
    )(x, w1, r1w)


def _tc_stage2(psum1, r1, b1, g1, be1, w2m, r2w):
    def body(p_ref, r1_ref, b1_ref, g1_ref, be1_ref, w2m_ref, r2w_ref,
             hw_ref, r2_ref, crec_ref):
        p = p_ref[0, :N] + p_ref[1, :N]
        c = jnp.maximum(p[:, P:P + 1], 1.0)
        z = p[:, :P] / c + r1_ref[...] + b1_ref[...]
        h1 = jax.nn.relu(_bn(z, g1_ref[...], be1_ref[...]))
        hw_ref[...] = jnp.dot(h1, w2m_ref[...],
                              preferred_element_type=jnp.float32)
        r2_ref[...] = jnp.dot(h1, r2w_ref[...],
                              preferred_element_type=jnp.float32)
        crec_ref[...] = jnp.broadcast_to(1.0 / c, (N, 8))

    return pl.pallas_call(
        body,
        out_shape=[jax.ShapeDtypeStruct((N, K2 * P), jnp.float32),
                   jax.ShapeDtypeStruct((N, P), jnp.float32),
                   jax.ShapeDtypeStruct((N, 8), jnp.float32)],
    )(psum1, r1, b1, g1, be1, w2m, r2w)


def _tc_stage3(psum2, r2, crec, b2, g2, be2):
    def body(p_ref, r2_ref, crec_ref, b2_ref, g2_ref, be2_ref, h2_ref):
        s = p_ref[0, :N] + p_ref[1, :N]
        z = s * crec_ref[:, :1] + r2_ref[...] + b2_ref[...]
        h2_ref[...] = jax.nn.relu(_bn(z, g2_ref[...], be2_ref[...]))

    return pl.pallas_call(
        body,
        out_shape=jax.ShapeDtypeStruct((N, P), jnp.float32),
    )(psum2, r2, crec, b2, g2, be2)


def _tc_stage4(psum3, crec, h2, x, w3, r3w, b3, g3, be3):
    def body(p_ref, crec_ref, h2_ref, x_ref, w3_ref, r3w_ref,
             b3_ref, g3_ref, be3_ref, out_ref):
        agg = (p_ref[0, :N] + p_ref[1, :N]) * crec_ref[:, :1]
        z = (jnp.dot(agg, w3_ref[...], preferred_element_type=jnp.float32)
             + jnp.dot(h2_ref[...], r3w_ref[...],
                       preferred_element_type=jnp.float32)
             + b3_ref[...])
        out_ref[...] = jax.nn.relu(_bn(z, g3_ref[...], be3_ref[...])
                                   + x_ref[...])

    return pl.pallas_call(
        body,
        out_shape=jax.ShapeDtypeStruct((N, D_IN), jnp.float32),
    )(psum3, crec, h2, x, w3, r3w, b3, g3, be3)


@jax.jit
def kernel(x, edge_index, edge_attr, W1, R1, b1, g1, be1,
           W2, R2, b2, g2, be2, W3, R3, b3, g3, be3):
    pad = EP - E
    src3 = jnp.pad(edge_index[0], (0, pad)).reshape(NW, NCH, CH)
    dst3 = jnp.pad(edge_index[1], (0, pad),
                   constant_values=N).reshape(NW, NCH, CH)
    f03 = jnp.pad(edge_attr[:, 0], (0, pad)).reshape(NW, NCH, CH)
    f13 = jnp.pad(edge_attr[:, 1], (0, pad)).reshape(NW, NCH, CH)

    b1r = b1.reshape(1, P)
    g1r = g1.reshape(1, P)
    be1r = be1.reshape(1, P)
    b2r = b2.reshape(1, P)
    g2r = g2.reshape(1, P)
    be2r = be2.reshape(1, P)
    b3r = b3.reshape(1, D_IN)
    g3r = g3.reshape(1, D_IN)
    be3r = be3.reshape(1, D_IN)

    t1, r1 = _tc_stage1(x, W1[0], R1)
    psum1 = _segsum(t1, src3, dst3, P + 8)
    w2m = jnp.transpose(W2, (1, 0, 2)).reshape(P, K2 * P)
    hw, r2, crec = _tc_stage2(psum1, r1, b1r, g1r, be1r, w2m, R2)

    hw_flat = hw.reshape(N * K2, P)
    psum2 = _spline_agg(hw_flat, src3, dst3, f03, f13)
    h2 = _tc_stage3(psum2, r2, crec, b2r, g2r, be2r)

    psum3 = _segsum(h2, src3, dst3, P)
    return _tc_stage4(psum3, crec, h2, x, W3[0], R3, b3r, g3r, be3r)
